# baseline (speedup 1.0000x reference)
import jax
import jax.numpy as jnp
from jax.experimental import pallas as pl
from jax.experimental.pallas import tpu as pltpu

_IN_F, _H1_F, _H2_F = 245, 120, 84
_H1_P, _H2_P = 128, 128


def _mlp_fused_body(x_ref, w1_ref, b1_ref, w2_ref, b2_ref, w3_ref, b3_ref,
                    o_ref):
    h1 = jnp.tanh(
        jnp.dot(x_ref[...], w1_ref[...], preferred_element_type=jnp.float32)
        + b1_ref[...]
    )
    h2 = jnp.tanh(
        jnp.dot(h1, w2_ref[...], preferred_element_type=jnp.float32)
        + b2_ref[...]
    )
    z = jax.lax.dot_general(
        w3_ref[...], h2, (((1,), (1,)), ((), ())),
        preferred_element_type=jnp.float32,
    )
    y = jnp.maximum(z + b3_ref[...], 0.0)
    o_ref[...] = y.reshape(o_ref.shape).astype(o_ref.dtype)


def _round_up(n, m):
    return ((n + m - 1) // m) * m


def kernel(x, w1, b1, w2, b2, w3, b3, *, tb=1024):
    B = x.shape[0]

    xb = x.astype(jnp.bfloat16)

    w1p = jnp.pad(w1, ((0, 0), (0, _H1_P - _H1_F))).astype(jnp.bfloat16)
    b1p = jnp.pad(b1, ((0, 0), (0, _H1_P - _H1_F)))
    w2p = jnp.pad(w2, ((0, _H1_P - _H1_F), (0, _H2_P - _H2_F)))
    b2p = jnp.pad(b2, ((0, 0), (0, _H2_P - _H2_F)))
    w3t = jnp.pad(w3.T, ((0, 0), (0, _H2_P - _H2_F)))

    TB = min(tb, _round_up(B, 128))
    B_pad = _round_up(B, TB)
    xp = xb if B_pad == B else jnp.pad(xb, ((0, B_pad - B), (0, 0)))

    out = pl.pallas_call(
        _mlp_fused_body,
        out_shape=jax.ShapeDtypeStruct((B_pad // 128, 128), jnp.float32),
        grid=(B_pad // TB,),
        in_specs=[
            pl.BlockSpec((TB, _IN_F), lambda i: (i, 0)),
            pl.BlockSpec((_IN_F, _H1_P), lambda i: (0, 0)),
            pl.BlockSpec((1, _H1_P), lambda i: (0, 0)),
            pl.BlockSpec((_H1_P, _H2_P), lambda i: (0, 0)),
            pl.BlockSpec((1, _H2_P), lambda i: (0, 0)),
            pl.BlockSpec((1, _H2_P), lambda i: (0, 0)),
            pl.BlockSpec((1, 1), lambda i: (0, 0)),
        ],
        out_specs=pl.BlockSpec((TB // 128, 128), lambda i: (i, 0)),
        compiler_params=pltpu.CompilerParams(
            dimension_semantics=("parallel",)
        ),
    )(xp, w1p, b1p, w2p, b2p, w3t, b3)

    return out.reshape(B_pad, 1)[:B]


# speedup vs baseline: 2.0035x; 2.0035x over previous
import jax
import jax.numpy as jnp
from jax.experimental import pallas as pl
from jax.experimental.pallas import tpu as pltpu

_IN_F, _H1_F, _H2_F = 245, 120, 84
_H1_P, _H2_P = 128, 128


def _mlp_body(xt_ref, w1_ref, b1_ref, w2_ref, b2_ref, w3_ref, b3_ref, o_ref):
    h1 = jnp.tanh(
        jnp.dot(w1_ref[...], xt_ref[...], preferred_element_type=jnp.float32)
        + b1_ref[...]
    )
    h2 = jnp.tanh(
        jnp.dot(w2_ref[...], h1, preferred_element_type=jnp.float32)
        + b2_ref[...]
    )
    y = jnp.maximum(
        jnp.dot(w3_ref[...], h2, preferred_element_type=jnp.float32)
        + b3_ref[...],
        0.0,
    )
    o_ref[...] = y.reshape(o_ref.shape).astype(o_ref.dtype)


def _round_up(n, m):
    return ((n + m - 1) // m) * m


def kernel(x, w1, b1, w2, b2, w3, b3, *, tb=1024):
    B = x.shape[0]

    xt = x.T

    w1t = jnp.pad(w1.T, ((0, _H1_P - _H1_F), (0, 0)))
    b1t = jnp.pad(b1.T, ((0, _H1_P - _H1_F), (0, 0)))
    w2t = jnp.pad(w2.T, ((0, _H2_P - _H2_F), (0, _H1_P - _H1_F)))
    b2t = jnp.pad(b2.T, ((0, _H2_P - _H2_F), (0, 0)))
    w3t = jnp.pad(w3.T, ((0, 0), (0, _H2_P - _H2_F)))

    TB = min(tb, _round_up(B, 128))
    B_pad = _round_up(B, TB)
    xtp = xt if B_pad == B else jnp.pad(xt, ((0, 0), (0, B_pad - B)))

    out = pl.pallas_call(
        _mlp_body,
        out_shape=jax.ShapeDtypeStruct((B_pad // 128, 128), jnp.float32),
        grid=(B_pad // TB,),
        in_specs=[
            pl.BlockSpec((_IN_F, TB), lambda i: (0, i)),
            pl.BlockSpec((_H1_P, _IN_F), lambda i: (0, 0)),
            pl.BlockSpec((_H1_P, 1), lambda i: (0, 0)),
            pl.BlockSpec((_H2_P, _H1_P), lambda i: (0, 0)),
            pl.BlockSpec((_H2_P, 1), lambda i: (0, 0)),
            pl.BlockSpec((1, _H2_P), lambda i: (0, 0)),
            pl.BlockSpec((1, 1), lambda i: (0, 0)),
        ],
        out_specs=pl.BlockSpec((TB // 128, 128), lambda i: (i, 0)),
        compiler_params=pltpu.CompilerParams(
            dimension_semantics=("arbitrary",)
        ),
    )(xtp, w1t, b1t, w2t, b2t, w3t, b3)

    return out.reshape(B_pad, 1)[:B]


# speedup vs baseline: 2.9429x; 1.4689x over previous
import jax
import jax.numpy as jnp
from jax.experimental import pallas as pl
from jax.experimental.pallas import tpu as pltpu

_IN_F, _H1_F, _H2_F = 245, 120, 84
_H1_P, _H2_P = 128, 128


def _mlp_body(xt_ref, w1_ref, b1_ref, w2_ref, b2_ref, w3_ref, b3_ref, o_ref):
    h1 = jnp.tanh(
        jnp.dot(w1_ref[...], xt_ref[...], preferred_element_type=jnp.float32)
        + b1_ref[...]
    )
    h2 = jnp.tanh(
        jnp.dot(w2_ref[...], h1, preferred_element_type=jnp.float32)
        + b2_ref[...]
    )
    y = jnp.maximum(
        jnp.dot(w3_ref[...], h2, preferred_element_type=jnp.float32)
        + b3_ref[...],
        0.0,
    )
    o_ref[...] = y.reshape(o_ref.shape).astype(o_ref.dtype)


def _round_up(n, m):
    return ((n + m - 1) // m) * m


def kernel(x, w1, b1, w2, b2, w3, b3, *, tb=2048):
    B = x.shape[0]

    xt = x.T

    w1t = jnp.pad(w1.T, ((0, _H1_P - _H1_F), (0, 0)))
    b1t = jnp.pad(b1.T, ((0, _H1_P - _H1_F), (0, 0)))
    w2t = jnp.pad(w2.T, ((0, _H2_P - _H2_F), (0, _H1_P - _H1_F)))
    b2t = jnp.pad(b2.T, ((0, _H2_P - _H2_F), (0, 0)))
    w3t = jnp.pad(w3.T, ((0, 0), (0, _H2_P - _H2_F)))

    TB = min(tb, _round_up(B, 128))
    B_pad = _round_up(B, TB)
    xtp = xt if B_pad == B else jnp.pad(xt, ((0, 0), (0, B_pad - B)))

    out = pl.pallas_call(
        _mlp_body,
        out_shape=jax.ShapeDtypeStruct((B_pad // 128, 128), jnp.float32),
        grid=(B_pad // TB,),
        in_specs=[
            pl.BlockSpec((_IN_F, TB), lambda i: (0, i)),
            pl.BlockSpec((_H1_P, _IN_F), lambda i: (0, 0)),
            pl.BlockSpec((_H1_P, 1), lambda i: (0, 0)),
            pl.BlockSpec((_H2_P, _H1_P), lambda i: (0, 0)),
            pl.BlockSpec((_H2_P, 1), lambda i: (0, 0)),
            pl.BlockSpec((1, _H2_P), lambda i: (0, 0)),
            pl.BlockSpec((1, 1), lambda i: (0, 0)),
        ],
        out_specs=pl.BlockSpec((TB // 128, 128), lambda i: (i, 0)),
        compiler_params=pltpu.CompilerParams(
            dimension_semantics=("arbitrary",)
        ),
    )(xtp, w1t, b1t, w2t, b2t, w3t, b3)

    return out.reshape(B_pad, 1)[:B]


# speedup vs baseline: 3.7577x; 1.2769x over previous
import jax
import jax.numpy as jnp
from jax.experimental import pallas as pl
from jax.experimental.pallas import tpu as pltpu

_IN_F, _H1_F, _H2_F = 245, 120, 84
_H1_P, _H2_P = 128, 128


def _mlp_body(xt_ref, w1_ref, b1_ref, w2_ref, b2_ref, w3_ref, b3_ref, o_ref):
    h1 = jnp.tanh(
        jnp.dot(w1_ref[...], xt_ref[...], preferred_element_type=jnp.float32)
        + b1_ref[...]
    )
    h2 = jnp.tanh(
        jnp.dot(w2_ref[...], h1, preferred_element_type=jnp.float32)
        + b2_ref[...]
    )
    y = jnp.maximum(
        jnp.dot(w3_ref[...], h2, preferred_element_type=jnp.float32)
        + b3_ref[...],
        0.0,
    )
    o_ref[...] = y.reshape(o_ref.shape).astype(o_ref.dtype)


def _round_up(n, m):
    return ((n + m - 1) // m) * m


def kernel(x, w1, b1, w2, b2, w3, b3, *, tb=4096):
    B = x.shape[0]

    xt = x.T

    w1t = jnp.pad(w1.T, ((0, _H1_P - _H1_F), (0, 0)))
    b1t = jnp.pad(b1.T, ((0, _H1_P - _H1_F), (0, 0)))
    w2t = jnp.pad(w2.T, ((0, _H2_P - _H2_F), (0, _H1_P - _H1_F)))
    b2t = jnp.pad(b2.T, ((0, _H2_P - _H2_F), (0, 0)))
    w3t = jnp.pad(w3.T, ((0, 0), (0, _H2_P - _H2_F)))

    TB = min(tb, _round_up(B, 128))
    B_pad = _round_up(B, TB)
    xtp = xt if B_pad == B else jnp.pad(xt, ((0, 0), (0, B_pad - B)))

    out = pl.pallas_call(
        _mlp_body,
        out_shape=jax.ShapeDtypeStruct((B_pad // 128, 128), jnp.float32),
        grid=(B_pad // TB,),
        in_specs=[
            pl.BlockSpec((_IN_F, TB), lambda i: (0, i)),
            pl.BlockSpec((_H1_P, _IN_F), lambda i: (0, 0)),
            pl.BlockSpec((_H1_P, 1), lambda i: (0, 0)),
            pl.BlockSpec((_H2_P, _H1_P), lambda i: (0, 0)),
            pl.BlockSpec((_H2_P, 1), lambda i: (0, 0)),
            pl.BlockSpec((1, _H2_P), lambda i: (0, 0)),
            pl.BlockSpec((1, 1), lambda i: (0, 0)),
        ],
        out_specs=pl.BlockSpec((TB // 128, 128), lambda i: (i, 0)),
        compiler_params=pltpu.CompilerParams(
            dimension_semantics=("arbitrary",)
        ),
    )(xtp, w1t, b1t, w2t, b2t, w3t, b3)

    return out.reshape(B_pad, 1)[:B]


# speedup vs baseline: 4.1710x; 1.1100x over previous
import jax
import jax.numpy as jnp
from jax.experimental import pallas as pl
from jax.experimental.pallas import tpu as pltpu

_IN_F, _H1_F, _H2_F = 245, 120, 84
_H1_P, _H2_P = 128, 128


def _mlp_body(xt_ref, w1_ref, b1_ref, w2_ref, b2_ref, w3_ref, b3_ref, o_ref):
    h1 = jnp.tanh(
        jnp.dot(w1_ref[...], xt_ref[...], preferred_element_type=jnp.float32)
        + b1_ref[...]
    )
    h2 = jnp.tanh(
        jnp.dot(w2_ref[...], h1, preferred_element_type=jnp.float32)
        + b2_ref[...]
    )
    y = jnp.maximum(
        jnp.dot(w3_ref[...], h2, preferred_element_type=jnp.float32)
        + b3_ref[...],
        0.0,
    )
    o_ref[...] = y.reshape(o_ref.shape).astype(o_ref.dtype)


def _round_up(n, m):
    return ((n + m - 1) // m) * m


def kernel(x, w1, b1, w2, b2, w3, b3, *, tb=8192):
    B = x.shape[0]

    xt = x.T

    w1t = jnp.pad(w1.T, ((0, _H1_P - _H1_F), (0, 0)))
    b1t = jnp.pad(b1.T, ((0, _H1_P - _H1_F), (0, 0)))
    w2t = jnp.pad(w2.T, ((0, _H2_P - _H2_F), (0, _H1_P - _H1_F)))
    b2t = jnp.pad(b2.T, ((0, _H2_P - _H2_F), (0, 0)))
    w3t = jnp.pad(w3.T, ((0, 0), (0, _H2_P - _H2_F)))

    TB = min(tb, _round_up(B, 128))
    B_pad = _round_up(B, TB)
    xtp = xt if B_pad == B else jnp.pad(xt, ((0, 0), (0, B_pad - B)))

    out = pl.pallas_call(
        _mlp_body,
        out_shape=jax.ShapeDtypeStruct((B_pad // 128, 128), jnp.float32),
        grid=(B_pad // TB,),
        in_specs=[
            pl.BlockSpec((_IN_F, TB), lambda i: (0, i)),
            pl.BlockSpec((_H1_P, _IN_F), lambda i: (0, 0)),
            pl.BlockSpec((_H1_P, 1), lambda i: (0, 0)),
            pl.BlockSpec((_H2_P, _H1_P), lambda i: (0, 0)),
            pl.BlockSpec((_H2_P, 1), lambda i: (0, 0)),
            pl.BlockSpec((1, _H2_P), lambda i: (0, 0)),
            pl.BlockSpec((1, 1), lambda i: (0, 0)),
        ],
        out_specs=pl.BlockSpec((TB // 128, 128), lambda i: (i, 0)),
        compiler_params=pltpu.CompilerParams(
            dimension_semantics=("arbitrary",)
        ),
    )(xtp, w1t, b1t, w2t, b2t, w3t, b3)

    return out.reshape(B_pad, 1)[:B]


# speedup vs baseline: 4.2080x; 1.0089x over previous
import jax
import jax.numpy as jnp
from jax.experimental import pallas as pl
from jax.experimental.pallas import tpu as pltpu

_IN_F, _H1_F, _H2_F = 245, 120, 84
_H1_P, _H2_P = 128, 128


def _mlp_body(xt_ref, w1_ref, b1_ref, w2_ref, b2_ref, w3_ref, b3_ref, o_ref):
    h1 = jnp.tanh(
        jnp.dot(w1_ref[...], xt_ref[...], preferred_element_type=jnp.float32)
        + b1_ref[...]
    )
    h2 = jnp.tanh(
        jnp.dot(w2_ref[...], h1, preferred_element_type=jnp.float32)
        + b2_ref[...]
    )
    y = jnp.maximum(
        jnp.dot(w3_ref[...], h2, preferred_element_type=jnp.float32)
        + b3_ref[...],
        0.0,
    )
    o_ref[...] = y.reshape(o_ref.shape).astype(o_ref.dtype)


def _round_up(n, m):
    return ((n + m - 1) // m) * m


def kernel(x, w1, b1, w2, b2, w3, b3, *, tb=16384):
    B = x.shape[0]

    xt = x.T

    w1t = jnp.pad(w1.T, ((0, _H1_P - _H1_F), (0, 0)))
    b1t = jnp.pad(b1.T, ((0, _H1_P - _H1_F), (0, 0)))
    w2t = jnp.pad(w2.T, ((0, _H2_P - _H2_F), (0, _H1_P - _H1_F)))
    b2t = jnp.pad(b2.T, ((0, _H2_P - _H2_F), (0, 0)))
    w3t = jnp.pad(w3.T, ((0, 0), (0, _H2_P - _H2_F)))

    TB = min(tb, _round_up(B, 128))
    B_pad = _round_up(B, TB)
    xtp = xt if B_pad == B else jnp.pad(xt, ((0, 0), (0, B_pad - B)))

    out = pl.pallas_call(
        _mlp_body,
        out_shape=jax.ShapeDtypeStruct((B_pad // 128, 128), jnp.float32),
        grid=(B_pad // TB,),
        in_specs=[
            pl.BlockSpec((_IN_F, TB), lambda i: (0, i)),
            pl.BlockSpec((_H1_P, _IN_F), lambda i: (0, 0)),
            pl.BlockSpec((_H1_P, 1), lambda i: (0, 0)),
            pl.BlockSpec((_H2_P, _H1_P), lambda i: (0, 0)),
            pl.BlockSpec((_H2_P, 1), lambda i: (0, 0)),
            pl.BlockSpec((1, _H2_P), lambda i: (0, 0)),
            pl.BlockSpec((1, 1), lambda i: (0, 0)),
        ],
        out_specs=pl.BlockSpec((TB // 128, 128), lambda i: (i, 0)),
        compiler_params=pltpu.CompilerParams(
            dimension_semantics=("arbitrary",)
        ),
    )(xtp, w1t, b1t, w2t, b2t, w3t, b3)

    return out.reshape(B_pad, 1)[:B]


# speedup vs baseline: 4.2256x; 1.0042x over previous
import jax
import jax.numpy as jnp
from jax.experimental import pallas as pl
from jax.experimental.pallas import tpu as pltpu

_IN_F, _H1_F, _H2_F = 245, 120, 84
_H1_P, _H2_P = 128, 128


def _mlp_body(xt_ref, w1_ref, b1_ref, w2_ref, b2_ref, w3_ref, b3_ref, o_ref):
    xb = xt_ref[...].astype(jnp.bfloat16)
    h1 = jnp.tanh(
        jnp.dot(w1_ref[...], xb, preferred_element_type=jnp.float32)
        + b1_ref[...]
    )
    h2 = jnp.tanh(
        jnp.dot(w2_ref[...], h1.astype(jnp.bfloat16),
                preferred_element_type=jnp.float32)
        + b2_ref[...]
    )
    y = jnp.maximum(
        jnp.dot(w3_ref[...], h2.astype(jnp.bfloat16),
                preferred_element_type=jnp.float32)
        + b3_ref[...],
        0.0,
    )
    o_ref[...] = y.reshape(o_ref.shape).astype(o_ref.dtype)


def _round_up(n, m):
    return ((n + m - 1) // m) * m


def kernel(x, w1, b1, w2, b2, w3, b3, *, tb=8192):
    B = x.shape[0]

    xt = x.T

    w1t = jnp.pad(w1.T, ((0, _H1_P - _H1_F), (0, 0)))
    w1t = w1t.astype(jnp.bfloat16)
    b1t = jnp.pad(b1.T, ((0, _H1_P - _H1_F), (0, 0)))
    w2t = jnp.pad(w2.T, ((0, _H2_P - _H2_F), (0, _H1_P - _H1_F)))
    w2t = w2t.astype(jnp.bfloat16)
    b2t = jnp.pad(b2.T, ((0, _H2_P - _H2_F), (0, 0)))
    w3t = jnp.pad(w3.T, ((0, 0), (0, _H2_P - _H2_F)))
    w3t = w3t.astype(jnp.bfloat16)

    TB = min(tb, _round_up(B, 128))
    B_pad = _round_up(B, TB)
    xtp = xt if B_pad == B else jnp.pad(xt, ((0, 0), (0, B_pad - B)))

    out = pl.pallas_call(
        _mlp_body,
        out_shape=jax.ShapeDtypeStruct((B_pad // 128, 128), jnp.float32),
        grid=(B_pad // TB,),
        in_specs=[
            pl.BlockSpec((_IN_F, TB), lambda i: (0, i)),
            pl.BlockSpec((_H1_P, _IN_F), lambda i: (0, 0)),
            pl.BlockSpec((_H1_P, 1), lambda i: (0, 0)),
            pl.BlockSpec((_H2_P, _H1_P), lambda i: (0, 0)),
            pl.BlockSpec((_H2_P, 1), lambda i: (0, 0)),
            pl.BlockSpec((1, _H2_P), lambda i: (0, 0)),
            pl.BlockSpec((1, 1), lambda i: (0, 0)),
        ],
        out_specs=pl.BlockSpec((TB // 128, 128), lambda i: (i, 0)),
        compiler_params=pltpu.CompilerParams(
            dimension_semantics=("arbitrary",)
        ),
    )(xtp, w1t, b1t, w2t, b2t, w3t, b3)

    return out.reshape(B_pad, 1)[:B]
